# baseline (speedup 1.0000x reference)
import functools

import jax
import jax.numpy as jnp
import numpy as np
from jax import lax
from jax.experimental import pallas as pl
from jax.experimental.pallas import tpu as pltpu
from jax.experimental.pallas import tpu_sc as plsc

_NUM_SCRATCHES = 20
_MAX_LENGTH = 50
_COLOR = 1.0
_NC, _NS = 2, 16
_NW = _NC * _NS


def _scratch_points(N, H, W):
    key = jax.random.key(42)
    k1, k2, k3, k4 = jax.random.split(key, 4)
    x_start = jax.random.randint(k1, (N, _NUM_SCRATCHES), 0, W)
    y_start = jax.random.randint(k2, (N, _NUM_SCRATCHES), 0, H)
    lengths = jax.random.randint(k3, (N, _NUM_SCRATCHES), 1, _MAX_LENGTH + 1)
    lengths = lengths.astype(jnp.float32)
    angles = jax.random.uniform(k4, (N, _NUM_SCRATCHES)) * 2 * 3.14159
    x_end = x_start.astype(jnp.float32) + lengths * jnp.cos(angles)
    y_end = y_start.astype(jnp.float32) + lengths * jnp.sin(angles)
    steps = int(_MAX_LENGTH * 1.5)
    t = jnp.linspace(0.0, 1.0, steps).reshape(1, 1, steps)
    xs = x_start.astype(jnp.float32)[..., None]
    ys = y_start.astype(jnp.float32)[..., None]
    xe = x_end[..., None]
    ye = y_end[..., None]
    x_points = (xs * (1 - t) + xe * t).astype(jnp.int32)
    y_points = (ys * (1 - t) + ye * t).astype(jnp.int32)
    x_points = jnp.clip(x_points, 0, W - 1).reshape(N, -1)
    y_points = jnp.clip(y_points, 0, H - 1).reshape(N, -1)
    return x_points, y_points


@functools.cache
def _word_scatter_lists(N, H, W):
    with jax.ensure_compile_time_eval():
        xp, yp = _scratch_points(N, H, W)
        x1 = jnp.clip(xp + 1, 0, W - 1)
        y1 = jnp.clip(yp + 1, 0, H - 1)
        pix = jnp.concatenate(
            [yp * W + xp, y1 * W + xp, yp * W + x1], axis=1)
        pix = np.asarray(pix)
    wpi = H * W // 4
    widx_l, wval_l = [], []
    for n in range(N):
        vals = np.zeros((wpi,), np.int64)
        p = np.unique(pix[n])
        np.bitwise_or.at(vals, p // 4, np.int64(1) << (8 * (p % 4)))
        nz = np.nonzero(vals)[0]
        widx_l.append(nz.astype(np.int32))
        wval_l.append(vals[nz].astype(np.uint32).astype(np.int64))
    kmax = -(-max(len(a) for a in widx_l) // 16) * 16
    widx = np.zeros((N, kmax), np.int32)
    wval = np.zeros((N, kmax), np.int64)
    for n in range(N):
        k = len(widx_l[n])
        widx[n, :k] = widx_l[n]
        wval[n, :k] = wval_l[n]
        widx[n, k:] = widx_l[n][-1]
        wval[n, k:] = wval_l[n][-1]
    wval_i32 = wval.astype(np.uint32).view(np.int32)
    return jnp.asarray(widx), jnp.asarray(wval_i32)


def _sc_build_mask(N, H, W, widx, wval):
    wpi = H * W // 4
    kmax = widx.shape[1]
    mesh = plsc.VectorSubcoreMesh(core_axis_name="c", subcore_axis_name="s")

    @functools.partial(
        pl.kernel,
        out_type=jax.ShapeDtypeStruct((N * wpi,), jnp.int32),
        mesh=mesh,
        compiler_params=pltpu.CompilerParams(needs_layout_passes=False),
        scratch_types=[
            pltpu.VMEM((wpi,), jnp.int32),
            pltpu.VMEM((kmax,), jnp.int32),
            pltpu.VMEM((kmax,), jnp.int32),
            pltpu.SemaphoreType.DMA,
            pltpu.SemaphoreType.DMA,
        ],
    )
    def mask_sc(widx_hbm, wval_hbm, out_hbm, mask_v, idx_v, val_v, si, sv):
        wid = lax.axis_index("s") * _NC + lax.axis_index("c")
        cp_i = pltpu.async_copy(widx_hbm.at[wid], idx_v, si)
        cp_v = pltpu.async_copy(wval_hbm.at[wid], val_v, sv)

        zeros16 = jnp.zeros((16,), jnp.int32)

        def zero_body(i, carry):
            mask_v[pl.ds(i * 16, 16)] = zeros16
            return carry

        lax.fori_loop(0, wpi // 16, zero_body, 0)
        cp_i.wait()
        cp_v.wait()

        for j in range(kmax // 16):
            wi = idx_v[pl.ds(j * 16, 16)]
            vv = val_v[pl.ds(j * 16, 16)]
            plsc.store_scatter(mask_v, [wi], vv)
        pltpu.sync_copy(mask_v, out_hbm.at[pl.ds(wid * wpi, wpi)])

    return mask_sc(widx, wval)


def kernel(img):
    N, C, H, W = img.shape
    widx, wval = _word_scatter_lists(N, H, W)
    mask_words = _sc_build_mask(N, H, W, widx, wval)
    mask = mask_words.reshape(N, H, W // 4)

    def body(img_ref, mask_ref, out_ref):
        m = mask_ref[0]
        m4 = jnp.repeat(m, 4, axis=1)
        shift = (lax.broadcasted_iota(jnp.int32, (H, W), 1) & 3) * 8
        sel = ((m4 >> shift) & 0xFF) != 0
        out_ref[0] = jnp.where(sel[None], jnp.float32(_COLOR), img_ref[0])

    return pl.pallas_call(
        body,
        grid=(N,),
        in_specs=[
            pl.BlockSpec((1, C, H, W), lambda n: (n, 0, 0, 0)),
            pl.BlockSpec((1, H, W // 4), lambda n: (n, 0, 0)),
        ],
        out_specs=pl.BlockSpec((1, C, H, W), lambda n: (n, 0, 0, 0)),
        out_shape=jax.ShapeDtypeStruct((N, C, H, W), jnp.float32),
    )(img, mask)


# speedup vs baseline: 3.1534x; 3.1534x over previous
import functools

import jax
import jax.numpy as jnp
import numpy as np
from jax import lax
from jax.experimental import pallas as pl
from jax.experimental.pallas import tpu as pltpu
from jax.experimental.pallas import tpu_sc as plsc

_NUM_SCRATCHES = 20
_MAX_LENGTH = 50
_COLOR = 1.0
_NC, _NS = 2, 16
_NW = _NC * _NS


def _scratch_points(N, H, W):
    key = jax.random.key(42)
    k1, k2, k3, k4 = jax.random.split(key, 4)
    x_start = jax.random.randint(k1, (N, _NUM_SCRATCHES), 0, W)
    y_start = jax.random.randint(k2, (N, _NUM_SCRATCHES), 0, H)
    lengths = jax.random.randint(k3, (N, _NUM_SCRATCHES), 1, _MAX_LENGTH + 1)
    lengths = lengths.astype(jnp.float32)
    angles = jax.random.uniform(k4, (N, _NUM_SCRATCHES)) * 2 * 3.14159
    x_end = x_start.astype(jnp.float32) + lengths * jnp.cos(angles)
    y_end = y_start.astype(jnp.float32) + lengths * jnp.sin(angles)
    steps = int(_MAX_LENGTH * 1.5)
    t = jnp.linspace(0.0, 1.0, steps).reshape(1, 1, steps)
    xs = x_start.astype(jnp.float32)[..., None]
    ys = y_start.astype(jnp.float32)[..., None]
    xe = x_end[..., None]
    ye = y_end[..., None]
    x_points = (xs * (1 - t) + xe * t).astype(jnp.int32)
    y_points = (ys * (1 - t) + ye * t).astype(jnp.int32)
    x_points = jnp.clip(x_points, 0, W - 1).reshape(N, -1)
    y_points = jnp.clip(y_points, 0, H - 1).reshape(N, -1)
    return x_points, y_points


@functools.cache
def _word_scatter_lists(N, H, W):
    with jax.ensure_compile_time_eval():
        xp, yp = _scratch_points(N, H, W)
        x1 = jnp.clip(xp + 1, 0, W - 1)
        y1 = jnp.clip(yp + 1, 0, H - 1)
        pix = jnp.concatenate(
            [yp * W + xp, y1 * W + xp, yp * W + x1], axis=1)
        pix = np.asarray(pix)
    wpi = H * W // 4
    widx_l, wval_l = [], []
    for n in range(N):
        vals = np.zeros((wpi,), np.int64)
        p = np.unique(pix[n])
        np.bitwise_or.at(vals, p // 4, np.int64(1) << (8 * (p % 4)))
        nz = np.nonzero(vals)[0]
        widx_l.append(nz.astype(np.int32))
        wval_l.append(vals[nz].astype(np.uint32).astype(np.int64))
    kmax = -(-max(len(a) for a in widx_l) // 16) * 16
    widx = np.zeros((N, kmax), np.int32)
    wval = np.zeros((N, kmax), np.int64)
    for n in range(N):
        k = len(widx_l[n])
        widx[n, :k] = widx_l[n]
        wval[n, :k] = wval_l[n]
        widx[n, k:] = widx_l[n][-1]
        wval[n, k:] = wval_l[n][-1]
    wval_i32 = wval.astype(np.uint32).view(np.int32)
    return jnp.asarray(widx), jnp.asarray(wval_i32)


def _sc_build_mask(N, H, W, widx, wval):
    wpi = H * W // 4
    kmax = widx.shape[1]
    mesh = plsc.VectorSubcoreMesh(core_axis_name="c", subcore_axis_name="s")

    @functools.partial(
        pl.kernel,
        out_type=jax.ShapeDtypeStruct((N * wpi,), jnp.int32),
        mesh=mesh,
        compiler_params=pltpu.CompilerParams(needs_layout_passes=False),
        scratch_types=[
            pltpu.VMEM((wpi,), jnp.int32),
            pltpu.VMEM((kmax,), jnp.int32),
            pltpu.VMEM((kmax,), jnp.int32),
            pltpu.SemaphoreType.DMA,
            pltpu.SemaphoreType.DMA,
        ],
    )
    def mask_sc(widx_hbm, wval_hbm, out_hbm, mask_v, idx_v, val_v, si, sv):
        wid = lax.axis_index("s") * _NC + lax.axis_index("c")
        cp_i = pltpu.async_copy(widx_hbm.at[wid], idx_v, si)
        cp_v = pltpu.async_copy(wval_hbm.at[wid], val_v, sv)

        zeros16 = jnp.zeros((16,), jnp.int32)

        def zero_body(i, carry):
            mask_v[pl.ds(i * 16, 16)] = zeros16
            return carry

        lax.fori_loop(0, wpi // 16, zero_body, 0)
        cp_i.wait()
        cp_v.wait()

        for j in range(kmax // 16):
            wi = idx_v[pl.ds(j * 16, 16)]
            vv = val_v[pl.ds(j * 16, 16)]
            plsc.store_scatter(mask_v, [wi], vv)
        pltpu.sync_copy(mask_v, out_hbm.at[pl.ds(wid * wpi, wpi)])

    return mask_sc(widx, wval)


def kernel(img):
    N, C, H, W = img.shape
    widx, wval = _word_scatter_lists(N, H, W)
    mask_words = _sc_build_mask(N, H, W, widx, wval)
    mask = lax.bitcast_convert_type(mask_words, jnp.uint8).reshape(N, H, W)

    def body(img_ref, mask_ref, out_ref):
        m = mask_ref[0] != 0
        out_ref[0, 0] = jnp.where(m, jnp.float32(_COLOR), img_ref[0, 0])

    return pl.pallas_call(
        body,
        grid=(N, C),
        in_specs=[
            pl.BlockSpec((1, 1, H, W), lambda n, c: (n, c, 0, 0)),
            pl.BlockSpec((1, H, W), lambda n, c: (n, 0, 0)),
        ],
        out_specs=pl.BlockSpec((1, 1, H, W), lambda n, c: (n, c, 0, 0)),
        out_shape=jax.ShapeDtypeStruct((N, C, H, W), jnp.float32),
    )(img, mask)


# speedup vs baseline: 4.7225x; 1.4976x over previous
import functools

import jax
import jax.numpy as jnp
import numpy as np
from jax import lax
from jax.experimental import pallas as pl
from jax.experimental.pallas import tpu as pltpu
from jax.experimental.pallas import tpu_sc as plsc

_NUM_SCRATCHES = 20
_MAX_LENGTH = 50
_COLOR = 1.0
_NC, _NS = 2, 16
_NW = _NC * _NS


def _scratch_points(N, H, W):
    key = jax.random.key(42)
    k1, k2, k3, k4 = jax.random.split(key, 4)
    x_start = jax.random.randint(k1, (N, _NUM_SCRATCHES), 0, W)
    y_start = jax.random.randint(k2, (N, _NUM_SCRATCHES), 0, H)
    lengths = jax.random.randint(k3, (N, _NUM_SCRATCHES), 1, _MAX_LENGTH + 1)
    lengths = lengths.astype(jnp.float32)
    angles = jax.random.uniform(k4, (N, _NUM_SCRATCHES)) * 2 * 3.14159
    x_end = x_start.astype(jnp.float32) + lengths * jnp.cos(angles)
    y_end = y_start.astype(jnp.float32) + lengths * jnp.sin(angles)
    steps = int(_MAX_LENGTH * 1.5)
    t = jnp.linspace(0.0, 1.0, steps).reshape(1, 1, steps)
    xs = x_start.astype(jnp.float32)[..., None]
    ys = y_start.astype(jnp.float32)[..., None]
    xe = x_end[..., None]
    ye = y_end[..., None]
    x_points = (xs * (1 - t) + xe * t).astype(jnp.int32)
    y_points = (ys * (1 - t) + ye * t).astype(jnp.int32)
    x_points = jnp.clip(x_points, 0, W - 1).reshape(N, -1)
    y_points = jnp.clip(y_points, 0, H - 1).reshape(N, -1)
    return x_points, y_points


@functools.cache
def _word_scatter_lists(N, H, W):
    with jax.ensure_compile_time_eval():
        xp, yp = _scratch_points(N, H, W)
        x1 = jnp.clip(xp + 1, 0, W - 1)
        y1 = jnp.clip(yp + 1, 0, H - 1)
        pix = jnp.concatenate(
            [yp * W + xp, y1 * W + xp, yp * W + x1], axis=1)
        pix = np.asarray(pix)
    wpi = H * W // 4
    wq = W // 4
    widx_l, wval_l = [], []
    for n in range(N):
        vals = np.zeros((wpi,), np.int64)
        p = np.unique(pix[n])
        py, px = p // W, p % W
        np.bitwise_or.at(vals, py * wq + px % wq,
                         np.int64(1) << (8 * (px // wq)))
        nz = np.nonzero(vals)[0]
        widx_l.append(nz.astype(np.int32))
        wval_l.append(vals[nz].astype(np.uint32).astype(np.int64))
    kmax = -(-max(len(a) for a in widx_l) // 16) * 16
    widx = np.zeros((N, kmax), np.int32)
    wval = np.zeros((N, kmax), np.int64)
    for n in range(N):
        k = len(widx_l[n])
        widx[n, :k] = widx_l[n]
        wval[n, :k] = wval_l[n]
        widx[n, k:] = widx_l[n][-1]
        wval[n, k:] = wval_l[n][-1]
    wval_i32 = wval.astype(np.uint32).view(np.int32)
    return jnp.asarray(widx), jnp.asarray(wval_i32)


def _sc_build_mask(N, H, W, widx, wval):
    wpi = H * W // 4
    kmax = widx.shape[1]
    mesh = plsc.VectorSubcoreMesh(core_axis_name="c", subcore_axis_name="s")

    @functools.partial(
        pl.kernel,
        out_type=jax.ShapeDtypeStruct((N * wpi,), jnp.int32),
        mesh=mesh,
        compiler_params=pltpu.CompilerParams(needs_layout_passes=False),
        scratch_types=[
            pltpu.VMEM((wpi,), jnp.int32),
            pltpu.VMEM((kmax,), jnp.int32),
            pltpu.VMEM((kmax,), jnp.int32),
            pltpu.SemaphoreType.DMA,
            pltpu.SemaphoreType.DMA,
        ],
    )
    def mask_sc(widx_hbm, wval_hbm, out_hbm, mask_v, idx_v, val_v, si, sv):
        wid = lax.axis_index("s") * _NC + lax.axis_index("c")
        cp_i = pltpu.async_copy(widx_hbm.at[wid], idx_v, si)
        cp_v = pltpu.async_copy(wval_hbm.at[wid], val_v, sv)

        zeros16 = jnp.zeros((16,), jnp.int32)

        def zero_body(i, carry):
            mask_v[pl.ds(i * 16, 16)] = zeros16
            return carry

        lax.fori_loop(0, wpi // 16, zero_body, 0)
        cp_i.wait()
        cp_v.wait()

        for j in range(kmax // 16):
            wi = idx_v[pl.ds(j * 16, 16)]
            vv = val_v[pl.ds(j * 16, 16)]
            plsc.store_scatter(mask_v, [wi], vv)
        pltpu.sync_copy(mask_v, out_hbm.at[pl.ds(wid * wpi, wpi)])

    return mask_sc(widx, wval)


def kernel(img):
    N, C, H, W = img.shape
    widx, wval = _word_scatter_lists(N, H, W)
    mask_words = _sc_build_mask(N, H, W, widx, wval)
    wq = W // 4
    mask = mask_words.reshape(N, H, wq)

    def body(img_ref, mask_ref, out_ref):
        m = mask_ref[0]
        for k in range(4):
            sel = ((m >> (8 * k)) & 0xFF) != 0
            sl = slice(k * wq, (k + 1) * wq)
            out_ref[0, 0, :, sl] = jnp.where(
                sel, jnp.float32(_COLOR), img_ref[0, 0, :, sl])

    return pl.pallas_call(
        body,
        grid=(N, C),
        in_specs=[
            pl.BlockSpec((1, 1, H, W), lambda n, c: (n, c, 0, 0)),
            pl.BlockSpec((1, H, wq), lambda n, c: (n, 0, 0)),
        ],
        out_specs=pl.BlockSpec((1, 1, H, W), lambda n, c: (n, c, 0, 0)),
        out_shape=jax.ShapeDtypeStruct((N, C, H, W), jnp.float32),
    )(img, mask)


# speedup vs baseline: 5.3515x; 1.1332x over previous
import functools

import jax
import jax.numpy as jnp
import numpy as np
from jax import lax
from jax.experimental import pallas as pl
from jax.experimental.pallas import tpu as pltpu
from jax.experimental.pallas import tpu_sc as plsc

_NUM_SCRATCHES = 20
_MAX_LENGTH = 50
_COLOR = 1.0
_NC, _NS = 2, 16
_NW = _NC * _NS


def _scratch_points(N, H, W):
    key = jax.random.key(42)
    k1, k2, k3, k4 = jax.random.split(key, 4)
    x_start = jax.random.randint(k1, (N, _NUM_SCRATCHES), 0, W)
    y_start = jax.random.randint(k2, (N, _NUM_SCRATCHES), 0, H)
    lengths = jax.random.randint(k3, (N, _NUM_SCRATCHES), 1, _MAX_LENGTH + 1)
    lengths = lengths.astype(jnp.float32)
    angles = jax.random.uniform(k4, (N, _NUM_SCRATCHES)) * 2 * 3.14159
    x_end = x_start.astype(jnp.float32) + lengths * jnp.cos(angles)
    y_end = y_start.astype(jnp.float32) + lengths * jnp.sin(angles)
    steps = int(_MAX_LENGTH * 1.5)
    t = jnp.linspace(0.0, 1.0, steps).reshape(1, 1, steps)
    xs = x_start.astype(jnp.float32)[..., None]
    ys = y_start.astype(jnp.float32)[..., None]
    xe = x_end[..., None]
    ye = y_end[..., None]
    x_points = (xs * (1 - t) + xe * t).astype(jnp.int32)
    y_points = (ys * (1 - t) + ye * t).astype(jnp.int32)
    x_points = jnp.clip(x_points, 0, W - 1).reshape(N, -1)
    y_points = jnp.clip(y_points, 0, H - 1).reshape(N, -1)
    return x_points, y_points


@functools.cache
def _word_scatter_lists(N, H, W):
    with jax.ensure_compile_time_eval():
        xp, yp = _scratch_points(N, H, W)
        x1 = jnp.clip(xp + 1, 0, W - 1)
        y1 = jnp.clip(yp + 1, 0, H - 1)
        pix = jnp.concatenate(
            [yp * W + xp, y1 * W + xp, yp * W + x1], axis=1)
        pix = np.asarray(pix)
    wpi = H * W // 4
    wq = W // 4
    widx_l, wval_l = [], []
    for n in range(N):
        vals = np.zeros((wpi,), np.int64)
        p = np.unique(pix[n])
        py, px = p // W, p % W
        np.bitwise_or.at(vals, py * wq + px % wq,
                         np.int64(1) << (8 * (px // wq)))
        nz = np.nonzero(vals)[0]
        widx_l.append(nz.astype(np.int32))
        wval_l.append(vals[nz].astype(np.uint32).astype(np.int64))
    kmax = -(-max(len(a) for a in widx_l) // 16) * 16
    widx = np.zeros((N, kmax), np.int32)
    wval = np.zeros((N, kmax), np.int64)
    for n in range(N):
        k = len(widx_l[n])
        widx[n, :k] = widx_l[n]
        wval[n, :k] = wval_l[n]
        widx[n, k:] = widx_l[n][-1]
        wval[n, k:] = wval_l[n][-1]
    wval_i32 = wval.astype(np.uint32).view(np.int32)
    return jnp.asarray(widx), jnp.asarray(wval_i32)


def _sc_build_mask(N, H, W, widx, wval):
    wpi = H * W // 4
    kmax = widx.shape[1]
    mesh = plsc.VectorSubcoreMesh(core_axis_name="c", subcore_axis_name="s")

    @functools.partial(
        pl.kernel,
        out_type=jax.ShapeDtypeStruct((N * wpi,), jnp.int32),
        mesh=mesh,
        compiler_params=pltpu.CompilerParams(needs_layout_passes=False),
        scratch_types=[
            pltpu.VMEM((wpi,), jnp.int32),
            pltpu.VMEM((kmax,), jnp.int32),
            pltpu.VMEM((kmax,), jnp.int32),
            pltpu.SemaphoreType.DMA,
            pltpu.SemaphoreType.DMA,
        ],
    )
    def mask_sc(widx_hbm, wval_hbm, out_hbm, mask_v, idx_v, val_v, si, sv):
        wid = lax.axis_index("s") * _NC + lax.axis_index("c")
        cp_i = pltpu.async_copy(widx_hbm.at[wid], idx_v, si)
        cp_v = pltpu.async_copy(wval_hbm.at[wid], val_v, sv)

        zeros16 = jnp.zeros((16,), jnp.int32)

        def zero_body(i, carry):
            for t in range(8):
                mask_v[pl.ds(i * 128 + t * 16, 16)] = zeros16
            return carry

        lax.fori_loop(0, wpi // 128, zero_body, 0)
        cp_i.wait()
        cp_v.wait()

        for j in range(kmax // 16):
            wi = idx_v[pl.ds(j * 16, 16)]
            vv = val_v[pl.ds(j * 16, 16)]
            plsc.store_scatter(mask_v, [wi], vv)
        pltpu.sync_copy(mask_v, out_hbm.at[pl.ds(wid * wpi, wpi)])

    return mask_sc(widx, wval)


def kernel(img):
    N, C, H, W = img.shape
    widx, wval = _word_scatter_lists(N, H, W)
    mask_words = _sc_build_mask(N, H, W, widx, wval)
    wq = W // 4
    mask = mask_words.reshape(N, H, wq)

    def body(img_ref, mask_ref, out_ref):
        m = mask_ref[0]
        for k in range(4):
            sel = ((m >> (8 * k)) & 0xFF) != 0
            sl = slice(k * wq, (k + 1) * wq)
            out_ref[0, 0, :, sl] = jnp.where(
                sel, jnp.float32(_COLOR), img_ref[0, 0, :, sl])

    return pl.pallas_call(
        body,
        grid=(N, C),
        in_specs=[
            pl.BlockSpec((1, 1, H, W), lambda n, c: (n, c, 0, 0)),
            pl.BlockSpec((1, H, wq), lambda n, c: (n, 0, 0)),
        ],
        out_specs=pl.BlockSpec((1, 1, H, W), lambda n, c: (n, c, 0, 0)),
        out_shape=jax.ShapeDtypeStruct((N, C, H, W), jnp.float32),
    )(img, mask)
